# baseline (speedup 1.0000x reference)
import functools

import jax
import jax.numpy as jnp
from jax import lax
from jax.experimental import pallas as pl
from jax.experimental.pallas import tpu as pltpu
from jax.experimental.pallas import tpu_sc as plsc

NBUF = 3
CHUNK = 256


def kernel(input_ids, table):
    B0, S = input_ids.shape
    V, D = table.shape
    B = B0 * S

    info = plsc.get_sparse_core_info()
    NC, NS = info.num_cores, info.num_subcores
    NW = NC * NS

    b_per_w = B // NW
    n_chunks = b_per_w // CHUNK
    assert b_per_w * NW == B
    assert n_chunks * CHUNK == b_per_w and n_chunks >= 2 * NBUF

    idx2 = input_ids.reshape(NW, b_per_w).astype(jnp.int32)

    mesh = plsc.VectorSubcoreMesh(core_axis_name="c", subcore_axis_name="s")

    @functools.partial(
        pl.kernel,
        mesh=mesh,
        out_type=jax.ShapeDtypeStruct((B, D), jnp.float32),
        scratch_types=[
            pltpu.VMEM((b_per_w,), jnp.int32),
            pltpu.VMEM((NBUF, CHUNK, D), jnp.float32),
        ]
        + [pltpu.SemaphoreType.DMA] * (2 * NBUF),
    )
    def emb(idx_hbm, table_hbm, out_hbm, idx_v, rows_v, *sems):
        gsem, osem = sems[:NBUF], sems[NBUF:]
        wid = lax.axis_index("s") * NC + lax.axis_index("c")
        base = wid * b_per_w
        pltpu.sync_copy(idx_hbm.at[wid], idx_v)

        def gather(t, b):
            return pltpu.make_async_copy(
                table_hbm.at[idx_v.at[pl.ds(t * CHUNK, CHUNK)]], rows_v.at[b], gsem[b]
            )

        def store(t, b):
            return pltpu.make_async_copy(
                rows_v.at[b], out_hbm.at[pl.ds(base + t * CHUNK, CHUNK)], osem[b]
            )

        for b in range(NBUF):
            gather(b, b).start()

        n_groups = (n_chunks - NBUF) // NBUF

        def group(g, carry):
            for b in range(NBUF):
                t = g * NBUF + b
                gather(t, b).wait()
                store(t, b).start()
            for b in range(NBUF):
                t = g * NBUF + b
                store(t, b).wait()
                gather(t + NBUF, b).start()
            return carry

        lax.fori_loop(0, n_groups, group, 0)

        t0 = n_groups * NBUF
        rem = n_chunks - t0
        for r in range(NBUF):
            gather(t0 + r, r).wait()
            store(t0 + r, r).start()
        for r in range(NBUF, rem):
            b = r % NBUF
            store(t0 + r - NBUF, b).wait()
            gather(t0 + r, b).start()
            gather(t0 + r, b).wait()
            store(t0 + r, b).start()
        for r in range(max(rem - NBUF, 0), rem):
            store(t0 + r, r % NBUF).wait()

    out = emb(idx2, table)
    return out.reshape(B0, S, D)
